# baseline (speedup 1.0000x reference)
import jax
import jax.numpy as jnp
from jax import lax
from jax.experimental import pallas as pl
from jax.experimental.pallas import tpu as pltpu
from jax.experimental.pallas import tpu_sc as plsc

N_FIELDS = 26
VOCAB = 100000
D_MODEL = 64
BATCH = 4096
LANES = 16
NC, NS = 2, 16
NW = NC * NS
B_TOTAL = N_FIELDS * BATCH
CH = 128
CPW = B_TOTAL // (NW * CH)


BPW = CPW * CH


def _body(xs_hbm, tab_hbm, out_hbm, idx_v, rows_v, sem):
    wid = lax.axis_index("s") * NC + lax.axis_index("c")
    base = wid * BPW
    pltpu.sync_copy(xs_hbm.at[pl.ds(base, BPW)], idx_v)
    for ch in range(CPW):
        g = wid * CPW + ch
        fld = lax.shift_right_logical(g, 5)
        pltpu.async_copy(
            tab_hbm.at[fld].at[idx_v.at[pl.ds(ch * CH, CH)]],
            rows_v, sem).wait()
        pltpu.sync_copy(rows_v, out_hbm.at[pl.ds(base + ch * CH, CH)])


def kernel(xs, tables):
    xs_flat = xs.reshape(B_TOTAL)
    fn = pl.kernel(
        _body,
        mesh=plsc.VectorSubcoreMesh(core_axis_name="c", subcore_axis_name="s"),
        compiler_params=pltpu.CompilerParams(use_tc_tiling_on_sc=False),
        out_type=jax.ShapeDtypeStruct((B_TOTAL, D_MODEL), jnp.float32),
        scratch_types=[
            pltpu.VMEM((BPW,), jnp.int32),
            pltpu.VMEM((CH, D_MODEL), jnp.float32),
            pltpu.SemaphoreType.DMA,
        ],
    )
    out = fn(xs_flat, tables)
    return out.reshape(N_FIELDS, BATCH, D_MODEL)


# speedup vs baseline: 2.2996x; 2.2996x over previous
import jax
import jax.numpy as jnp
from jax import lax
from jax.experimental import pallas as pl
from jax.experimental.pallas import tpu as pltpu
from jax.experimental.pallas import tpu_sc as plsc

N_FIELDS = 26
VOCAB = 100000
D_MODEL = 64
BATCH = 4096
NC, NS, L = 2, 16, 16
NW = NC * NS
B_TOTAL = N_FIELDS * BATCH

SUP = 4096
W = 512
WPS = SUP // W
SPF = (VOCAB + SUP - 1) // SUP
NSUP = N_FIELDS * SPF
ROUNDS = (NSUP + NW - 1) // NW
VFULL = (VOCAB // W) * W
TAIL0 = (VOCAB // 128) * 128
WCLAMP = TAIL0 - W
NTAIL = VOCAB - TAIL0
RING = 32


def _wait_row(out_hbm, rows_v, osem):
    pltpu.make_async_copy(
        rows_v.at[pl.ds(0, D_MODEL)],
        out_hbm.at[pl.ds(0, D_MODEL)], osem).wait()


def _body(xs_hbm, tab_hbm, tail_hbm, out_hbm,
          idx_f, tabw, tailw, hitv, hitb, subv, subb, rows_v, sem, osem):
    wid = lax.axis_index("s") * NC + lax.axis_index("c")

    def emit_rows(cnt_w, outst, fbase, gather_col):
        def hit_body(j, o):
            vv = subv[pl.ds(j, L)][0]
            b = subb[pl.ds(j, L)][0]
            slot = lax.rem(j, RING) * D_MODEL
            for k in range(D_MODEL // L):
                col = gather_col(vv, k)
                rows_v[pl.ds(slot + k * L, L)] = col

            @pl.when(o >= RING)
            def _():
                _wait_row(out_hbm, rows_v, osem)

            pltpu.async_copy(
                rows_v.at[pl.ds(slot, D_MODEL)],
                out_hbm.at[pl.ds(fbase + b * D_MODEL, D_MODEL)], osem)
            return jnp.where(o >= RING, o, o + 1)

        return lax.fori_loop(0, cnt_w, hit_body, outst)

    def sub_scan(cnt, wlo, whi):
        def sbody(jv, c):
            v = hitv[pl.ds(jv * L, L)]
            b = hitb[pl.ds(jv * L, L)]
            m = (v >= wlo) & (v < whi)
            pc = plsc.all_reduce_population_count(m)
            plsc.store_compressed(subv.at[pl.ds(c, L)], v, mask=m)
            plsc.store_compressed(subb.at[pl.ds(c, L)], b, mask=m)
            return c + pc[0]

        nv = lax.div(cnt + (L - 1), L)
        return lax.fori_loop(0, nv, sbody, 0)

    def round_fn(sid, outst):
        f = lax.div(sid, SPF)
        si = lax.rem(sid, SPF)
        s0 = si * SUP
        fbase = f * (BATCH * D_MODEL)
        islast = si == (SPF - 1)

        pltpu.sync_copy(xs_hbm.at[pl.ds(f * BATCH, BATCH)], idx_f)

        def scan_body(k, c):
            v = idx_f[pl.ds(k * L, L)]
            m = (v >= s0) & (v < s0 + SUP)
            pc = plsc.all_reduce_population_count(m)
            plsc.store_compressed(hitv.at[pl.ds(c, L)], v, mask=m)
            plsc.store_compressed(
                hitb.at[pl.ds(c, L)], lax.iota(jnp.int32, L) + k * L, mask=m)
            return c + pc[0]

        cnt = lax.fori_loop(0, BATCH // L, scan_body, 0)

        for k in range(WPS):
            wlo = s0 + k * W
            w0c = jnp.minimum(wlo, WCLAMP)
            whi = jnp.minimum(wlo + W, TAIL0)
            for tr in range(8):
                pltpu.async_copy(
                    tab_hbm.at[f, pl.ds(tr * 8, 8), pl.ds(w0c, W)],
                    tabw.at[pl.ds(tr * 8, 8), :], sem)
            cnt_w = sub_scan(cnt, wlo, whi)
            for tr in range(8):
                pltpu.make_async_copy(
                    tab_hbm.at[f, pl.ds(tr * 8, 8), pl.ds(w0c, W)],
                    tabw.at[pl.ds(tr * 8, 8), :], sem).wait()

            def gather_win(vv, kk):
                d_vec = lax.iota(jnp.int32, L) + kk * L
                vv_vec = jnp.full((L,), vv, jnp.int32) - w0c
                return plsc.load_gather(tabw, [d_vec, vv_vec])

            outst = emit_rows(cnt_w, outst, fbase, gather_win)

        def tail_fn(o):
            pltpu.sync_copy(
                tail_hbm.at[pl.ds(f * (NTAIL * D_MODEL), NTAIL * D_MODEL)],
                tailw)
            cnt_t = sub_scan(cnt, TAIL0, VOCAB)

            def gather_tail(vv, kk):
                idx = (vv - TAIL0) * D_MODEL + lax.iota(jnp.int32, L) + kk * L
                return plsc.load_gather(tailw, [idx])

            return emit_rows(cnt_t, o, fbase, gather_tail)

        return lax.cond(islast, tail_fn, lambda o: o, outst)

    def one_round(r, outst):
        sid = wid + r * NW
        return lax.cond(sid < NSUP, round_fn, lambda s, o: o, sid, outst)

    outst = lax.fori_loop(0, ROUNDS, one_round, 0)

    def drain(j, c):
        _wait_row(out_hbm, rows_v, osem)
        return c

    lax.fori_loop(0, outst, drain, 0)


def kernel(xs, tables):
    xs_flat = xs.reshape(B_TOTAL)
    tab_v = jnp.swapaxes(tables, 1, 2)
    tail = tables[:, TAIL0:, :].reshape(-1)
    fn = pl.kernel(
        _body,
        mesh=plsc.VectorSubcoreMesh(core_axis_name="c", subcore_axis_name="s"),
        compiler_params=pltpu.CompilerParams(
            use_tc_tiling_on_sc=True, needs_layout_passes=False),
        out_type=jax.ShapeDtypeStruct((B_TOTAL * D_MODEL,), jnp.float32),
        scratch_types=[
            pltpu.VMEM((BATCH,), jnp.int32),
            pltpu.VMEM((D_MODEL, W), jnp.float32),
            pltpu.VMEM((NTAIL * D_MODEL,), jnp.float32),
            pltpu.VMEM((BATCH + L,), jnp.int32),
            pltpu.VMEM((BATCH + L,), jnp.int32),
            pltpu.VMEM((BATCH + L,), jnp.int32),
            pltpu.VMEM((BATCH + L,), jnp.int32),
            pltpu.VMEM((RING * D_MODEL,), jnp.float32),
            pltpu.SemaphoreType.DMA,
            pltpu.SemaphoreType.DMA,
        ],
    )
    out = fn(xs_flat, tab_v, tail)
    return out.reshape(N_FIELDS, BATCH, D_MODEL)


# speedup vs baseline: 3.2856x; 1.4287x over previous
import jax
import jax.numpy as jnp
from jax import lax
from jax.experimental import pallas as pl
from jax.experimental.pallas import tpu as pltpu
from jax.experimental.pallas import tpu_sc as plsc

N_FIELDS = 26
VOCAB = 100000
D_MODEL = 64
BATCH = 4096
NC, NS, L = 2, 16, 16
NW = NC * NS
B_TOTAL = N_FIELDS * BATCH

SUP = 4096
W = 512
WPS = SUP // W
SPF = (VOCAB + SUP - 1) // SUP
NSUP = N_FIELDS * SPF
ROUNDS = (NSUP + NW - 1) // NW
VFULL = (VOCAB // W) * W
TAIL0 = (VOCAB // 128) * 128
WCLAMP = TAIL0 - W
NTAIL = VOCAB - TAIL0
RING = 32


def _wait_row(out_hbm, rows_v, osem):
    pltpu.make_async_copy(
        rows_v.at[pl.ds(0, D_MODEL)],
        out_hbm.at[pl.ds(0, D_MODEL)], osem).wait()


def _body(xs_hbm, tab_hbm, tail_hbm, out_hbm,
          idx_f, tabw, tailw, hitv, hitb, subv, subb, rows_v, sem, osem):
    wid = lax.axis_index("s") * NC + lax.axis_index("c")

    def emit_rows(cnt_w, outst, fbase, gather_col):
        def hit_body(j, o):
            vv = subv[pl.ds(j, L)][0]
            b = subb[pl.ds(j, L)][0]
            slot = lax.rem(j, RING) * D_MODEL
            for k in range(D_MODEL // L):
                col = gather_col(vv, k)
                rows_v[pl.ds(slot + k * L, L)] = col

            @pl.when(o >= RING)
            def _():
                _wait_row(out_hbm, rows_v, osem)

            pltpu.async_copy(
                rows_v.at[pl.ds(slot, D_MODEL)],
                out_hbm.at[pl.ds(fbase + b * D_MODEL, D_MODEL)], osem)
            return jnp.where(o >= RING, o, o + 1)

        return lax.fori_loop(0, cnt_w, hit_body, outst)

    def sub_scan(cnt, wlo, whi):
        def sbody(jv, c):
            v = hitv[pl.ds(jv * L, L)]
            b = hitb[pl.ds(jv * L, L)]
            m = (v >= wlo) & (v < whi)
            pc = plsc.all_reduce_population_count(m)
            plsc.store_compressed(subv.at[pl.ds(c, L)], v, mask=m)
            plsc.store_compressed(subb.at[pl.ds(c, L)], b, mask=m)
            return c + pc[0]

        nv = lax.div(cnt + (L - 1), L)
        return lax.fori_loop(0, nv, sbody, 0)

    def wstart(f, k, s0):
        w0c = jnp.minimum(s0 + k * W, WCLAMP)
        for tr in range(8):
            pltpu.async_copy(
                tab_hbm.at[f, pl.ds(tr * 8, 8), pl.ds(w0c, W)],
                tabw.at[k & 1, pl.ds(tr * 8, 8), :], sem)
        return w0c

    def wwait(f, k, s0, w0c):
        for tr in range(8):
            pltpu.make_async_copy(
                tab_hbm.at[f, pl.ds(tr * 8, 8), pl.ds(w0c, W)],
                tabw.at[k & 1, pl.ds(tr * 8, 8), :], sem).wait()

    def round_fn(sid, outst):
        f = lax.div(sid, SPF)
        si = lax.rem(sid, SPF)
        s0 = si * SUP
        fbase = f * (BATCH * D_MODEL)
        islast = si == (SPF - 1)

        pltpu.sync_copy(xs_hbm.at[pl.ds(f * BATCH, BATCH)], idx_f)
        w0c = wstart(f, 0, s0)

        def scan_body(k, c):
            v = idx_f[pl.ds(k * L, L)]
            m = lax.bitcast_convert_type(v - s0, jnp.uint32) < jnp.uint32(SUP)
            pc = plsc.all_reduce_population_count(m)
            plsc.store_compressed(hitv.at[pl.ds(c, L)], v, mask=m)
            plsc.store_compressed(
                hitb.at[pl.ds(c, L)], lax.iota(jnp.int32, L) + k * L, mask=m)
            return c + pc[0]

        cnt = lax.fori_loop(0, BATCH // L, scan_body, 0)

        for k in range(WPS):
            wlo = s0 + k * W
            whi = jnp.minimum(wlo + W, TAIL0)
            cnt_w = sub_scan(cnt, wlo, whi)
            wwait(f, k, s0, w0c)
            if k + 1 < WPS:
                w1c = wstart(f, k + 1, s0)

            def gather_win(vv, kk, _w0c=w0c, _k=k):
                d_vec = lax.iota(jnp.int32, L) + kk * L
                vv_vec = jnp.full((L,), vv, jnp.int32) - _w0c
                return plsc.load_gather(tabw.at[_k & 1], [d_vec, vv_vec])

            outst = emit_rows(cnt_w, outst, fbase, gather_win)
            if k + 1 < WPS:
                w0c = w1c

        def tail_fn(o):
            pltpu.sync_copy(
                tail_hbm.at[pl.ds(f * (NTAIL * D_MODEL), NTAIL * D_MODEL)],
                tailw)
            cnt_t = sub_scan(cnt, TAIL0, VOCAB)

            def gather_tail(vv, kk):
                idx = (vv - TAIL0) * D_MODEL + lax.iota(jnp.int32, L) + kk * L
                return plsc.load_gather(tailw, [idx])

            return emit_rows(cnt_t, o, fbase, gather_tail)

        return lax.cond(islast, tail_fn, lambda o: o, outst)

    def one_round(r, outst):
        sid = wid + r * NW
        return lax.cond(sid < NSUP, round_fn, lambda s, o: o, sid, outst)

    outst = lax.fori_loop(0, ROUNDS, one_round, 0)

    def drain(j, c):
        _wait_row(out_hbm, rows_v, osem)
        return c

    lax.fori_loop(0, outst, drain, 0)


def kernel(xs, tables):
    xs_flat = xs.reshape(B_TOTAL)
    tab_v = jnp.swapaxes(tables, 1, 2)
    tail = tables[:, TAIL0:, :].reshape(-1)
    fn = pl.kernel(
        _body,
        mesh=plsc.VectorSubcoreMesh(core_axis_name="c", subcore_axis_name="s"),
        compiler_params=pltpu.CompilerParams(
            use_tc_tiling_on_sc=True, needs_layout_passes=False),
        out_type=jax.ShapeDtypeStruct((B_TOTAL * D_MODEL,), jnp.float32),
        scratch_types=[
            pltpu.VMEM((BATCH,), jnp.int32),
            pltpu.VMEM((2, D_MODEL, W), jnp.float32),
            pltpu.VMEM((NTAIL * D_MODEL,), jnp.float32),
            pltpu.VMEM((BATCH + L,), jnp.int32),
            pltpu.VMEM((BATCH + L,), jnp.int32),
            pltpu.VMEM((BATCH + L,), jnp.int32),
            pltpu.VMEM((BATCH + L,), jnp.int32),
            pltpu.VMEM((RING * D_MODEL,), jnp.float32),
            pltpu.SemaphoreType.DMA,
            pltpu.SemaphoreType.DMA,
        ],
    )
    out = fn(xs_flat, tab_v, tail)
    return out.reshape(N_FIELDS, BATCH, D_MODEL)
